# baseline (speedup 1.0000x reference)
import jax
import jax.numpy as jnp
import numpy as np
from jax.experimental import pallas as pl
from jax.experimental.pallas import tpu as pltpu

B, C, H, W, P = 128, 128, 26, 26, 128

_INV = np.asarray([
    184, 208, 48, 67, 25, 52, 83, 24, 113, 126, 92, 93, 241, 150, 173, 101,
    41, 175, 90, 46, 107, 191, 82, 181, 116, 143, 196, 235, 210, 32, 63, 14,
    98, 169, 50, 1, 218, 44, 176, 71, 128, 189, 64, 111, 39, 5, 255, 168,
    215, 103, 105, 230, 151, 95, 87, 228, 72, 200, 42, 135, 166, 47, 158, 17,
    190, 23, 142, 69, 248, 80, 68, 130, 58, 149, 236, 139, 164, 88, 28, 170,
    81, 117, 22, 36, 244, 16, 177, 163, 156, 203, 53, 226, 112, 134, 97, 253,
    119, 141, 233, 10, 225, 26, 27, 229, 252, 182, 123, 234, 35, 243, 57, 45,
    15, 211, 20, 209, 195, 18, 85, 220, 222, 0, 224, 43, 194, 207, 206, 232,
    124, 37, 2, 140, 162, 188, 242, 121, 237, 104, 106, 8, 114, 251, 49, 247,
    11, 185, 204, 89, 3, 250, 160, 221, 12, 62, 120, 59, 51, 30, 137, 100,
    122, 192, 214, 29, 132, 197, 193, 55, 198, 74, 216, 246, 213, 77, 19, 54,
    6, 34, 60, 7, 171, 202, 205, 31, 102, 110, 65, 129, 9, 13, 179, 125,
    73, 199, 155, 76, 144, 4, 165, 84, 127, 136, 153, 152, 239, 245, 146, 133,
    201, 161, 138, 40, 38, 186, 254, 231, 249, 99, 75, 61, 183, 240, 227, 70,
    223, 212, 187, 217, 174, 238, 159, 178, 180, 115, 94, 86, 96, 108, 148, 118,
    33, 79, 145, 147, 167, 78, 66, 172, 131, 91, 157, 56, 219, 109, 21, 154,
], dtype=np.int32)


def _transform(x):
    a = jnp.swapaxes(x, 0, 1)
    a = a.reshape(H * C, W)
    b = jnp.transpose(a, (1, 0))
    return b.reshape(W, H, C)


def _body(inv_ref, g_ref, lt_ref, ltp_ref, ltn_ref,
          x1_ref, x2_ref, x1l_ref, x2l_ref, tgt_ref, tl_ref, csem):
    s = pl.program_id(0)
    j = pl.program_id(1)

    @pl.when(j == 0)
    def _():
        t = _transform(lt_ref[0])
        tl_ref[...] = t
        x2_ref[0] = _transform(ltp_ref[0])
        tgt_ref[0] = jnp.ones((H, W), jnp.float32)

    @pl.when(j == 1)
    def _():
        x2_ref[0] = _transform(ltn_ref[0])
        tgt_ref[0] = jnp.zeros((H, W), jnp.float32)

    cp_l = pltpu.make_async_copy(tl_ref, x1l_ref.at[0], csem)
    cp_l.start()
    x1_ref[0] = jnp.broadcast_to(g_ref[s, :][None, None, :], (W, H, P))
    cp_l.wait()
    cp2 = pltpu.make_async_copy(x2_ref.at[0], x2l_ref.at[0], csem)
    cp2.start()
    cp2.wait()


@jax.jit
def kernel(global_t, local_t_map, local_t_prev_map, local_t_n_map):
    inv = jnp.asarray(_INV)

    src_map = lambda s, j, inv_ref: (s, 0, 0, 0)
    dst_map = lambda s, j, inv_ref: (inv_ref[s + B * j], 0, 0, 0)

    grid_spec = pltpu.PrefetchScalarGridSpec(
        num_scalar_prefetch=1,
        grid=(B, 2),
        in_specs=[
            pl.BlockSpec((B, P), lambda s, j, inv_ref: (0, 0)),
            pl.BlockSpec((1, C, H, W), src_map),
            pl.BlockSpec((1, C, H, W), src_map),
            pl.BlockSpec((1, C, H, W), src_map),
        ],
        out_specs=[
            pl.BlockSpec((1, W, H, P), dst_map),
            pl.BlockSpec((1, W, H, C), dst_map),
            pl.BlockSpec((1, W, H, C), dst_map),
            pl.BlockSpec((1, W, H, C), dst_map),
            pl.BlockSpec((1, H, W), lambda s, j, inv_ref: (inv_ref[s + B * j], 0, 0)),
        ],
        scratch_shapes=[
            pltpu.VMEM((W, H, C), jnp.float32),
            pltpu.SemaphoreType.DMA,
        ],
    )

    x1, x2, x1_l, x2_l, target = pl.pallas_call(
        _body,
        grid_spec=grid_spec,
        out_shape=[
            jax.ShapeDtypeStruct((2 * B, W, H, P), jnp.float32),
            jax.ShapeDtypeStruct((2 * B, W, H, C), jnp.float32),
            jax.ShapeDtypeStruct((2 * B, W, H, C), jnp.float32),
            jax.ShapeDtypeStruct((2 * B, W, H, C), jnp.float32),
            jax.ShapeDtypeStruct((2 * B, H, W), jnp.float32),
        ],
    )(inv, global_t, local_t_map, local_t_prev_map, local_t_n_map)

    return (x1, x2, x1_l, x2_l, target)


# speedup vs baseline: 2.1961x; 2.1961x over previous
import jax
import jax.numpy as jnp
import numpy as np
from jax.experimental import pallas as pl
from jax.experimental.pallas import tpu as pltpu

B, C, H, W, P = 128, 128, 26, 26, 128
HW = H * W

_INV = np.asarray([
    184, 208, 48, 67, 25, 52, 83, 24, 113, 126, 92, 93, 241, 150, 173, 101,
    41, 175, 90, 46, 107, 191, 82, 181, 116, 143, 196, 235, 210, 32, 63, 14,
    98, 169, 50, 1, 218, 44, 176, 71, 128, 189, 64, 111, 39, 5, 255, 168,
    215, 103, 105, 230, 151, 95, 87, 228, 72, 200, 42, 135, 166, 47, 158, 17,
    190, 23, 142, 69, 248, 80, 68, 130, 58, 149, 236, 139, 164, 88, 28, 170,
    81, 117, 22, 36, 244, 16, 177, 163, 156, 203, 53, 226, 112, 134, 97, 253,
    119, 141, 233, 10, 225, 26, 27, 229, 252, 182, 123, 234, 35, 243, 57, 45,
    15, 211, 20, 209, 195, 18, 85, 220, 222, 0, 224, 43, 194, 207, 206, 232,
    124, 37, 2, 140, 162, 188, 242, 121, 237, 104, 106, 8, 114, 251, 49, 247,
    11, 185, 204, 89, 3, 250, 160, 221, 12, 62, 120, 59, 51, 30, 137, 100,
    122, 192, 214, 29, 132, 197, 193, 55, 198, 74, 216, 246, 213, 77, 19, 54,
    6, 34, 60, 7, 171, 202, 205, 31, 102, 110, 65, 129, 9, 13, 179, 125,
    73, 199, 155, 76, 144, 4, 165, 84, 127, 136, 153, 152, 239, 245, 146, 133,
    201, 161, 138, 40, 38, 186, 254, 231, 249, 99, 75, 61, 183, 240, 227, 70,
    223, 212, 187, 217, 174, 238, 159, 178, 180, 115, 94, 86, 96, 108, 148, 118,
    33, 79, 145, 147, 167, 78, 66, 172, 131, 91, 157, 56, 219, 109, 21, 154,
], dtype=np.int32)


def _transpose_whc(x):
    t = x.T.reshape(H, W, C)
    t = jnp.transpose(t, (1, 0, 2))
    return t.reshape(W * H, C)


def _body(inv_ref, g_ref, lt_ref, ltp_ref, ltn_ref,
          x1_ref, x2_ref, x1l_ref, tgt_ref, tl_ref):
    s = pl.program_id(0)
    j = pl.program_id(1)

    @pl.when(j == 0)
    def _():
        tl = _transpose_whc(lt_ref[0])
        tl_ref[...] = tl
        x1l_ref[0] = tl
        x2_ref[0] = _transpose_whc(ltp_ref[0])
        tgt_ref[0] = jnp.ones((H, W), jnp.float32)

    @pl.when(j == 1)
    def _():
        x1l_ref[0] = tl_ref[...]
        x2_ref[0] = _transpose_whc(ltn_ref[0])
        tgt_ref[0] = jnp.zeros((H, W), jnp.float32)

    x1_ref[0] = jnp.broadcast_to(g_ref[s, :][None, :], (HW, P))


@jax.jit
def kernel(global_t, local_t_map, local_t_prev_map, local_t_n_map):
    lt = local_t_map.reshape(B, C, HW)
    ltp = local_t_prev_map.reshape(B, C, HW)
    ltn = local_t_n_map.reshape(B, C, HW)
    inv = jnp.asarray(_INV)

    src_map = lambda s, j, inv_ref: (s, 0, 0)
    src_map_n = lambda s, j, inv_ref: (jnp.where(j == 1, s, jnp.maximum(s - 1, 0)), 0, 0)
    dst_map = lambda s, j, inv_ref: (inv_ref[s + B * j], 0, 0)

    grid_spec = pltpu.PrefetchScalarGridSpec(
        num_scalar_prefetch=1,
        grid=(B, 2),
        in_specs=[
            pl.BlockSpec((B, P), lambda s, j, inv_ref: (0, 0)),
            pl.BlockSpec((1, C, HW), src_map),
            pl.BlockSpec((1, C, HW), src_map),
            pl.BlockSpec((1, C, HW), src_map_n),
        ],
        out_specs=[
            pl.BlockSpec((1, HW, P), dst_map),
            pl.BlockSpec((1, HW, C), dst_map),
            pl.BlockSpec((1, HW, C), dst_map),
            pl.BlockSpec((1, H, W), lambda s, j, inv_ref: (inv_ref[s + B * j], 0, 0)),
        ],
        scratch_shapes=[pltpu.VMEM((HW, C), jnp.float32)],
    )

    x1f, x2f, x1lf, target = pl.pallas_call(
        _body,
        grid_spec=grid_spec,
        out_shape=[
            jax.ShapeDtypeStruct((2 * B, HW, P), jnp.float32),
            jax.ShapeDtypeStruct((2 * B, HW, C), jnp.float32),
            jax.ShapeDtypeStruct((2 * B, HW, C), jnp.float32),
            jax.ShapeDtypeStruct((2 * B, H, W), jnp.float32),
        ],
    )(inv, global_t, lt, ltp, ltn)

    x1 = x1f.reshape(2 * B, W, H, P)
    x2 = x2f.reshape(2 * B, W, H, C)
    x1_l = x1lf.reshape(2 * B, W, H, C)
    return (x1, x2, x1_l, x2, target)
